# Initial kernel scaffold; baseline (speedup 1.0000x reference)
import jax
import jax.numpy as jnp
from jax.experimental import pallas as pl


def kernel(x, edge_index, Wn1, Ws1, b1, g1, be1, Wn2, Ws2, b2, g2, be2, Wn3, Ws3, b3):
    raise NotImplementedError("write your pallas kernel here")


# speedup vs baseline: 3.0932x; 3.0932x over previous
import functools

import jax
import jax.numpy as jnp
from jax import lax
from jax.experimental import pallas as pl
from jax.experimental.pallas import tpu as pltpu
from jax.experimental.pallas import tpu_sc as plsc

N = 10000
E = 320000
D = 128
EPS = 1e-5

NC = 2
NS = 16
EPT = E // NS
B = 80
NB = EPT // B
HALF = 5120
HROWS = 5248
RPT = HROWS // NS

NPAD = 10240
RPT1 = NPAD // NS


def _zero_2d(buf, rows, cols):
    z16 = jnp.zeros((16,), jnp.float32)

    def zr(r, carry):
        for c8 in range(cols // 16):
            buf[r, pl.ds(c8 * 16, 16)] = z16
        return carry

    lax.fori_loop(0, rows, zr, 0)


def _zero_1d(buf, n):
    z16 = jnp.zeros((16,), jnp.float32)

    def zr(r, carry):
        buf[pl.ds(r * 16, 16)] = z16
        return carry

    lax.fori_loop(0, n // 16, zr, 0)


_MESH = plsc.VectorSubcoreMesh(core_axis_name="c", subcore_axis_name="s")


@functools.partial(
    pl.kernel,
    out_type=jax.ShapeDtypeStruct((NC * HROWS, D), jnp.float32),
    mesh=_MESH,
    scratch_types=(
        pltpu.VMEM((B,), jnp.int32),
        pltpu.VMEM((B,), jnp.int32),
        pltpu.VMEM((B, D), jnp.float32),
        pltpu.VMEM((RPT, D), jnp.float32),
        pltpu.VMEM_SHARED((HROWS, D), jnp.float32),
        pltpu.SemaphoreType.DMA,
    ),
)
def _sc_agg(h_hbm, src_hbm, dst_hbm, out_hbm,
            src_v, dst_v, rows_v, zbuf_v, acc_sh, sem):
    c = lax.axis_index("c")
    s = lax.axis_index("s")
    e0 = s * EPT
    r0 = s * RPT
    base = c * HALF

    _zero_2d(zbuf_v, RPT, D)
    pltpu.sync_copy(zbuf_v, acc_sh.at[pl.ds(r0, RPT)])
    plsc.subcore_barrier()

    def step(i, carry):
        b0 = e0 + i * B
        pltpu.sync_copy(src_hbm.at[pl.ds(b0, B)], src_v)
        pltpu.sync_copy(dst_hbm.at[pl.ds(b0, B)], dst_v)
        pltpu.async_copy(h_hbm.at[src_v], rows_v, sem).wait()
        for k in range(B // 16):
            d16 = dst_v[pl.ds(k * 16, 16)]
            loc = d16 - base
            ok = (loc >= 0) & (loc < HALF)
            dst_v[pl.ds(k * 16, 16)] = jnp.where(ok, loc, HALF)
        pltpu.sync_copy(rows_v, acc_sh.at[dst_v], add=True)
        return carry

    lax.fori_loop(0, NB, step, 0)
    plsc.subcore_barrier()

    o0 = c * HROWS + r0
    pltpu.sync_copy(acc_sh.at[pl.ds(r0, RPT)], zbuf_v)
    pltpu.sync_copy(zbuf_v, out_hbm.at[pl.ds(o0, RPT)])


@functools.partial(
    pl.kernel,
    out_type=jax.ShapeDtypeStruct((NC * NPAD,), jnp.float32),
    mesh=_MESH,
    scratch_types=(
        pltpu.VMEM((B,), jnp.int32),
        pltpu.VMEM((B,), jnp.int32),
        pltpu.VMEM((B,), jnp.float32),
        pltpu.VMEM((RPT1,), jnp.float32),
        pltpu.VMEM_SHARED((NPAD,), jnp.float32),
        pltpu.SemaphoreType.DMA,
    ),
)
def _sc_agg_scalar(t_hbm, src_hbm, dst_hbm, out_hbm,
                   src_v, dst_v, vals_v, zbuf_v, acc_sh, sem):
    c = lax.axis_index("c")
    s = lax.axis_index("s")
    epw = E // (NC * NS)
    e0 = (c * NS + s) * epw
    r0 = s * RPT1

    _zero_1d(zbuf_v, RPT1)
    pltpu.sync_copy(zbuf_v, acc_sh.at[pl.ds(r0, RPT1)])
    plsc.subcore_barrier()

    def step(i, carry):
        b0 = e0 + i * B
        pltpu.sync_copy(src_hbm.at[pl.ds(b0, B)], src_v)
        pltpu.sync_copy(dst_hbm.at[pl.ds(b0, B)], dst_v)
        pltpu.async_copy(t_hbm.at[src_v], vals_v, sem).wait()
        pltpu.sync_copy(vals_v, acc_sh.at[dst_v], add=True)
        return carry

    lax.fori_loop(0, epw // B, step, 0)
    plsc.subcore_barrier()

    o0 = c * NPAD + r0
    pltpu.sync_copy(acc_sh.at[pl.ds(r0, RPT1)], zbuf_v)
    pltpu.sync_copy(zbuf_v, out_hbm.at[pl.ds(o0, RPT1)])


def _dense_layer(agg, degp, h, Wn, Ws, b, g, be):
    def body(agg_ref, degp_ref, h_ref, Wn_ref, Ws_ref, b_ref, g_ref, be_ref,
             o_ref):
        deg = jnp.maximum(degp_ref[0] + degp_ref[1], 1.0)
        agg_m = agg_ref[...] / deg
        lin = (jnp.dot(agg_m, Wn_ref[...], preferred_element_type=jnp.float32)
               + jnp.dot(h_ref[...], Ws_ref[...],
                         preferred_element_type=jnp.float32)
               + b_ref[...])
        mu = jnp.mean(lin, axis=0, keepdims=True)
        cen = lin - mu
        var = jnp.mean(cen * cen, axis=0, keepdims=True)
        y = cen * lax.rsqrt(var + EPS) * g_ref[...] + be_ref[...]
        o_ref[...] = jnp.maximum(y, 0.0)

    return pl.pallas_call(
        body,
        out_shape=jax.ShapeDtypeStruct((N, D), jnp.float32),
    )(agg, degp, h, Wn, Ws, b, g, be)


def _proj_layer(h2, W3):
    def body(h_ref, W3_ref, ts_ref):
        ts_ref[...] = jnp.dot(h_ref[...], W3_ref[...],
                              preferred_element_type=jnp.float32)

    return pl.pallas_call(
        body,
        out_shape=jax.ShapeDtypeStruct((N, 2), jnp.float32),
    )(h2, W3)


def _final_layer(agg3p, degp, s, b3):
    def body(agg3p_ref, degp_ref, s_ref, b3_ref, o_ref):
        deg = jnp.maximum(degp_ref[0] + degp_ref[1], 1.0)
        lin = (agg3p_ref[0] + agg3p_ref[1]) / deg + s_ref[...] + b3_ref[...]
        o_ref[...] = jax.nn.sigmoid(lin)

    return pl.pallas_call(
        body,
        out_shape=jax.ShapeDtypeStruct((N, 1), jnp.float32),
    )(agg3p, degp, s, b3)


def _agg_full(table):
    halves = table.reshape(NC, HROWS, D)[:, :HALF]
    return halves.reshape(NC * HALF, D)[:N]


def kernel(x, edge_index, Wn1, Ws1, b1, g1, be1, Wn2, Ws2, b2, g2, be2,
           Wn3, Ws3, b3):
    src = edge_index[0].astype(jnp.int32)
    dst = edge_index[1].astype(jnp.int32)

    degp_flat = _sc_agg_scalar(jnp.ones((N,), jnp.float32), src, dst)
    degp = degp_flat.reshape(NC, NPAD, 1)[:, :N]

    agg1 = _agg_full(_sc_agg(x, src, dst))
    h1 = _dense_layer(agg1, degp, x, Wn1, Ws1, b1.reshape(1, D),
                      g1.reshape(1, D), be1.reshape(1, D))

    agg2 = _agg_full(_sc_agg(h1, src, dst))
    h2 = _dense_layer(agg2, degp, h1, Wn2, Ws2, b2.reshape(1, D),
                      g2.reshape(1, D), be2.reshape(1, D))

    W3 = jnp.concatenate([Wn3, Ws3], axis=1)
    ts = _proj_layer(h2, W3)
    t = ts[:, 0:1].reshape(N)
    s = ts[:, 1:2]

    agg3p = _sc_agg_scalar(t, src, dst).reshape(NC, NPAD, 1)[:, :N]

    out = _final_layer(agg3p, degp, s, b3.reshape(1, 1))
    return out.reshape(N)


# speedup vs baseline: 3.4251x; 1.1073x over previous
import functools

import jax
import jax.numpy as jnp
from jax import lax
from jax.experimental import pallas as pl
from jax.experimental.pallas import tpu as pltpu
from jax.experimental.pallas import tpu_sc as plsc

N = 10000
E = 320000
D = 128
EPS = 1e-5

NC = 2
NS = 16
EPT = E // NS
B = 80
NB = EPT // B
HALF = 5120
HROWS = 5248
RPT = HROWS // NS

NPAD = 10240
RPT1 = NPAD // NS


def _zero_2d(buf, rows, cols):
    z16 = jnp.zeros((16,), jnp.float32)

    def zr(r, carry):
        for c8 in range(cols // 16):
            buf[r, pl.ds(c8 * 16, 16)] = z16
        return carry

    lax.fori_loop(0, rows, zr, 0)


def _zero_1d(buf, n):
    z16 = jnp.zeros((16,), jnp.float32)

    def zr(r, carry):
        buf[pl.ds(r * 16, 16)] = z16
        return carry

    lax.fori_loop(0, n // 16, zr, 0)


_MESH = plsc.VectorSubcoreMesh(core_axis_name="c", subcore_axis_name="s")


@functools.partial(
    pl.kernel,
    out_type=(jax.ShapeDtypeStruct((NC * HROWS, D), jnp.float32),
              jax.ShapeDtypeStruct((NC * HROWS,), jnp.float32)),
    mesh=_MESH,
    scratch_types=(
        pltpu.VMEM((B,), jnp.int32),
        pltpu.VMEM((B,), jnp.int32),
        pltpu.VMEM((B, D), jnp.float32),
        pltpu.VMEM((RPT, D), jnp.float32),
        pltpu.VMEM((B,), jnp.float32),
        pltpu.VMEM((336,), jnp.float32),
        pltpu.VMEM_SHARED((HROWS, D), jnp.float32),
        pltpu.VMEM_SHARED((HROWS,), jnp.float32),
        pltpu.SemaphoreType.DMA,
    ),
)
def _sc_agg(h_hbm, src_hbm, dst_hbm, out_hbm, deg_hbm,
            src_v, dst_v, rows_v, zbuf_v, ones_v, dzero_v, acc_sh, deg_sh,
            sem):
    c = lax.axis_index("c")
    s = lax.axis_index("s")
    e0 = s * EPT
    r0 = s * RPT
    base = c * HALF

    _zero_2d(zbuf_v, RPT, D)
    _zero_1d(dzero_v, 336)
    one16 = jnp.full((16,), 1.0, jnp.float32)
    for k in range(B // 16):
        ones_v[pl.ds(k * 16, 16)] = one16
    pltpu.sync_copy(zbuf_v, acc_sh.at[pl.ds(r0, RPT)])
    pltpu.sync_copy(dzero_v.at[pl.ds(0, RPT)], deg_sh.at[pl.ds(r0, RPT)])
    plsc.subcore_barrier()

    def step(i, carry):
        b0 = e0 + i * B
        pltpu.sync_copy(src_hbm.at[pl.ds(b0, B)], src_v)
        pltpu.sync_copy(dst_hbm.at[pl.ds(b0, B)], dst_v)
        pltpu.async_copy(h_hbm.at[src_v], rows_v, sem).wait()
        for k in range(B // 16):
            d16 = dst_v[pl.ds(k * 16, 16)]
            loc = d16 - base
            ok = (loc >= 0) & (loc < HALF)
            dst_v[pl.ds(k * 16, 16)] = jnp.where(ok, loc, HALF)
        pltpu.sync_copy(rows_v, acc_sh.at[dst_v], add=True)
        pltpu.sync_copy(ones_v, deg_sh.at[dst_v], add=True)
        return carry

    lax.fori_loop(0, NB, step, 0)
    plsc.subcore_barrier()

    o0 = c * HROWS + r0
    pltpu.sync_copy(acc_sh.at[pl.ds(r0, RPT)], zbuf_v)
    pltpu.sync_copy(zbuf_v, out_hbm.at[pl.ds(o0, RPT)])
    pltpu.sync_copy(deg_sh.at[pl.ds(r0, RPT)], dzero_v.at[pl.ds(0, RPT)])
    pltpu.sync_copy(dzero_v.at[pl.ds(0, RPT)], deg_hbm.at[pl.ds(o0, RPT)])


@functools.partial(
    pl.kernel,
    out_type=jax.ShapeDtypeStruct((NC * NPAD,), jnp.float32),
    mesh=_MESH,
    scratch_types=(
        pltpu.VMEM((B,), jnp.int32),
        pltpu.VMEM((B,), jnp.int32),
        pltpu.VMEM((B,), jnp.float32),
        pltpu.VMEM((RPT1,), jnp.float32),
        pltpu.VMEM_SHARED((NPAD,), jnp.float32),
        pltpu.SemaphoreType.DMA,
    ),
)
def _sc_agg_scalar(t_hbm, src_hbm, dst_hbm, out_hbm,
                   src_v, dst_v, vals_v, zbuf_v, acc_sh, sem):
    c = lax.axis_index("c")
    s = lax.axis_index("s")
    epw = E // (NC * NS)
    e0 = (c * NS + s) * epw
    r0 = s * RPT1

    _zero_1d(zbuf_v, RPT1)
    pltpu.sync_copy(zbuf_v, acc_sh.at[pl.ds(r0, RPT1)])
    plsc.subcore_barrier()

    def step(i, carry):
        b0 = e0 + i * B
        pltpu.sync_copy(src_hbm.at[pl.ds(b0, B)], src_v)
        pltpu.sync_copy(dst_hbm.at[pl.ds(b0, B)], dst_v)
        pltpu.async_copy(t_hbm.at[src_v], vals_v, sem).wait()
        pltpu.sync_copy(vals_v, acc_sh.at[dst_v], add=True)
        return carry

    lax.fori_loop(0, epw // B, step, 0)
    plsc.subcore_barrier()

    o0 = c * NPAD + r0
    pltpu.sync_copy(acc_sh.at[pl.ds(r0, RPT1)], zbuf_v)
    pltpu.sync_copy(zbuf_v, out_hbm.at[pl.ds(o0, RPT1)])


def _dense_layer(agg, deg, h, Wn, Ws, b, g, be):
    def body(agg_ref, deg_ref, h_ref, Wn_ref, Ws_ref, b_ref, g_ref, be_ref,
             o_ref):
        deg = jnp.maximum(deg_ref[...], 1.0)
        agg_m = agg_ref[...] / deg
        lin = (jnp.dot(agg_m, Wn_ref[...], preferred_element_type=jnp.float32)
               + jnp.dot(h_ref[...], Ws_ref[...],
                         preferred_element_type=jnp.float32)
               + b_ref[...])
        mu = jnp.mean(lin, axis=0, keepdims=True)
        cen = lin - mu
        var = jnp.mean(cen * cen, axis=0, keepdims=True)
        y = cen * lax.rsqrt(var + EPS) * g_ref[...] + be_ref[...]
        o_ref[...] = jnp.maximum(y, 0.0)

    return pl.pallas_call(
        body,
        out_shape=jax.ShapeDtypeStruct((N, D), jnp.float32),
    )(agg, deg, h, Wn, Ws, b, g, be)


def _proj_layer(h2, W3):
    def body(h_ref, W3_ref, ts_ref):
        ts_ref[...] = jnp.dot(h_ref[...], W3_ref[...],
                              preferred_element_type=jnp.float32)

    return pl.pallas_call(
        body,
        out_shape=jax.ShapeDtypeStruct((N, 2), jnp.float32),
    )(h2, W3)


def _final_layer(agg3p, deg, s, b3):
    def body(agg3p_ref, deg_ref, s_ref, b3_ref, o_ref):
        deg = jnp.maximum(deg_ref[...], 1.0)
        lin = (agg3p_ref[0] + agg3p_ref[1]) / deg + s_ref[...] + b3_ref[...]
        o_ref[...] = jax.nn.sigmoid(lin)

    return pl.pallas_call(
        body,
        out_shape=jax.ShapeDtypeStruct((N, 1), jnp.float32),
    )(agg3p, deg, s, b3)


def _agg_full(table):
    halves = table.reshape(NC, HROWS, D)[:, :HALF]
    return halves.reshape(NC * HALF, D)[:N]


def _deg_full(degh):
    halves = degh.reshape(NC, HROWS)[:, :HALF]
    return halves.reshape(NC * HALF)[:N].reshape(N, 1)


def kernel(x, edge_index, Wn1, Ws1, b1, g1, be1, Wn2, Ws2, b2, g2, be2,
           Wn3, Ws3, b3):
    src = edge_index[0].astype(jnp.int32)
    dst = edge_index[1].astype(jnp.int32)

    agg1_raw, deg_raw = _sc_agg(x, src, dst)
    agg1 = _agg_full(agg1_raw)
    deg = _deg_full(deg_raw)
    h1 = _dense_layer(agg1, deg, x, Wn1, Ws1, b1.reshape(1, D),
                      g1.reshape(1, D), be1.reshape(1, D))

    agg2_raw, _ = _sc_agg(h1, src, dst)
    agg2 = _agg_full(agg2_raw)
    h2 = _dense_layer(agg2, deg, h1, Wn2, Ws2, b2.reshape(1, D),
                      g2.reshape(1, D), be2.reshape(1, D))

    W3 = jnp.concatenate([Wn3, Ws3], axis=1)
    ts = _proj_layer(h2, W3)
    t = ts[:, 0:1].reshape(N)
    s = ts[:, 1:2]

    agg3p = _sc_agg_scalar(t, src, dst).reshape(NC, NPAD, 1)[:, :N]

    out = _final_layer(agg3p, deg, s, b3.reshape(1, 1))
    return out.reshape(N)


# speedup vs baseline: 4.4800x; 1.3080x over previous
import functools

import jax
import jax.numpy as jnp
from jax import lax
from jax.experimental import pallas as pl
from jax.experimental.pallas import tpu as pltpu
from jax.experimental.pallas import tpu_sc as plsc

N = 10000
E = 320000
D = 128
EPS = 1e-5

NC = 2
NS = 16
EPT = E // NS
B = 80
NB = EPT // B
HALF = 5120
HROWS = 5248
RPT = HROWS // NS

NPAD = 10240
RPT1 = NPAD // NS


def _zero_2d(buf, rows, cols):
    z16 = jnp.zeros((16,), jnp.float32)

    def zr(r, carry):
        for c8 in range(cols // 16):
            buf[r, pl.ds(c8 * 16, 16)] = z16
        return carry

    lax.fori_loop(0, rows, zr, 0)


def _zero_1d(buf, n):
    z16 = jnp.zeros((16,), jnp.float32)

    def zr(r, carry):
        buf[pl.ds(r * 16, 16)] = z16
        return carry

    lax.fori_loop(0, n // 16, zr, 0)


_MESH = plsc.VectorSubcoreMesh(core_axis_name="c", subcore_axis_name="s")

NW = NC * NS
EPW = E // NW
CAP = 10112


@functools.partial(
    pl.kernel,
    out_type=(jax.ShapeDtypeStruct((2 * NW * CAP,), jnp.int32),
              jax.ShapeDtypeStruct((2 * NW * CAP,), jnp.int32),
              jax.ShapeDtypeStruct((2 * NW * 16,), jnp.int32)),
    mesh=_MESH,
    scratch_types=(
        pltpu.VMEM((B,), jnp.int32),
        pltpu.VMEM((B,), jnp.int32),
        pltpu.VMEM((CAP,), jnp.int32),
        pltpu.VMEM((CAP,), jnp.int32),
        pltpu.VMEM((CAP,), jnp.int32),
        pltpu.VMEM((CAP,), jnp.int32),
        pltpu.VMEM((16,), jnp.int32),
        pltpu.SemaphoreType.DMA,
    ),
    compiler_params=pltpu.CompilerParams(needs_layout_passes=False),
)
def _sc_partition(src_hbm, dst_hbm, srcs_out, dsts_out, cnts_out,
                  src_v, dst_v, srcA, dstA, srcB, dstB, cnt_v, sem):
    c = lax.axis_index("c")
    s = lax.axis_index("s")
    w = c * NS + s
    e0 = w * EPW
    lane = lax.iota(jnp.int32, 16)
    trash = jnp.int32(CAP - 16)

    def step(i, cnts):
        ca, cb = cnts
        b0 = e0 + i * B
        pltpu.sync_copy(src_hbm.at[pl.ds(b0, B)], src_v)
        pltpu.sync_copy(dst_hbm.at[pl.ds(b0, B)], dst_v)
        for k in range(B // 16):
            s16 = src_v[pl.ds(k * 16, 16)]
            d16 = dst_v[pl.ds(k * 16, 16)]
            mA = d16 < HALF
            miA = jnp.where(mA, jnp.int32(1), jnp.int32(0))
            posA = jnp.cumsum(miA) - miA
            idxA = jnp.where(mA, ca + posA, trash + lane)
            plsc.store_scatter(srcA, [idxA], s16)
            plsc.store_scatter(dstA, [idxA], d16)
            ca = ca + jnp.sum(miA)
            miB = jnp.int32(1) - miA
            posB = jnp.cumsum(miB) - miB
            idxB = jnp.where(mA, trash + lane, cb + posB)
            plsc.store_scatter(srcB, [idxB], s16)
            plsc.store_scatter(dstB, [idxB], d16 - HALF)
            cb = cb + jnp.sum(miB)
        return ca, cb

    ca, cb = lax.fori_loop(0, EPW // B, step, (jnp.int32(0), jnp.int32(0)))

    zero16 = jnp.zeros((16,), jnp.int32)
    garb16 = jnp.full((16,), HALF, jnp.int32)
    for k in range(B // 16):
        srcA[pl.ds(ca + k * 16, 16)] = zero16
        dstA[pl.ds(ca + k * 16, 16)] = garb16
        srcB[pl.ds(cb + k * 16, 16)] = zero16
        dstB[pl.ds(cb + k * 16, 16)] = garb16
    ca = ((ca + (B - 1)) // B) * B
    cb = ((cb + (B - 1)) // B) * B

    oA = w * CAP
    oB = (NW + w) * CAP
    pltpu.sync_copy(srcA, srcs_out.at[pl.ds(oA, CAP)])
    pltpu.sync_copy(dstA, dsts_out.at[pl.ds(oA, CAP)])
    pltpu.sync_copy(srcB, srcs_out.at[pl.ds(oB, CAP)])
    pltpu.sync_copy(dstB, dsts_out.at[pl.ds(oB, CAP)])
    cnt_v[pl.ds(0, 16)] = zero16 + ca
    pltpu.sync_copy(cnt_v, cnts_out.at[pl.ds(w * 16, 16)])
    cnt_v[pl.ds(0, 16)] = zero16 + cb
    pltpu.sync_copy(cnt_v, cnts_out.at[pl.ds((NW + w) * 16, 16)])


@functools.partial(
    pl.kernel,
    out_type=(jax.ShapeDtypeStruct((NC * HROWS, D), jnp.float32),
              jax.ShapeDtypeStruct((NC * HROWS,), jnp.float32)),
    mesh=_MESH,
    scratch_types=(
        pltpu.VMEM((B,), jnp.int32),
        pltpu.VMEM((B,), jnp.int32),
        pltpu.VMEM((B, D), jnp.float32),
        pltpu.VMEM((RPT, D), jnp.float32),
        pltpu.VMEM((B,), jnp.float32),
        pltpu.VMEM((336,), jnp.float32),
        pltpu.VMEM((16,), jnp.int32),
        pltpu.VMEM_SHARED((HROWS, D), jnp.float32),
        pltpu.VMEM_SHARED((HROWS,), jnp.float32),
        pltpu.SemaphoreType.DMA,
    ),
    compiler_params=pltpu.CompilerParams(needs_layout_passes=False),
)
def _sc_agg(h_hbm, srcs_hbm, dsts_hbm, cnts_hbm, out_hbm, deg_hbm,
            src_v, dst_v, rows_v, zbuf_v, ones_v, dzero_v, cnt_v, acc_sh,
            deg_sh, sem):
    c = lax.axis_index("c")
    s = lax.axis_index("s")
    r0 = s * RPT

    _zero_2d(zbuf_v, RPT, D)
    _zero_1d(dzero_v, 336)
    one16 = jnp.full((16,), 1.0, jnp.float32)
    for k in range(B // 16):
        ones_v[pl.ds(k * 16, 16)] = one16
    pltpu.sync_copy(zbuf_v, acc_sh.at[pl.ds(r0, RPT)])
    pltpu.sync_copy(dzero_v.at[pl.ds(0, RPT)], deg_sh.at[pl.ds(r0, RPT)])
    plsc.subcore_barrier()

    for j in range(2):
        w = s * 2 + j
        region = (c * NW + w) * CAP
        pltpu.sync_copy(cnts_hbm.at[pl.ds((c * NW + w) * 16, 16)], cnt_v)
        cnt = jnp.max(cnt_v[pl.ds(0, 16)])

        def step(i, carry):
            b0 = region + i * B
            pltpu.sync_copy(srcs_hbm.at[pl.ds(b0, B)], src_v)
            pltpu.sync_copy(dsts_hbm.at[pl.ds(b0, B)], dst_v)
            pltpu.async_copy(h_hbm.at[src_v], rows_v, sem).wait()
            pltpu.sync_copy(rows_v, acc_sh.at[dst_v], add=True)
            pltpu.sync_copy(ones_v, deg_sh.at[dst_v], add=True)
            return carry

        lax.fori_loop(0, cnt // B, step, 0)
    plsc.subcore_barrier()

    o0 = c * HROWS + r0
    pltpu.sync_copy(acc_sh.at[pl.ds(r0, RPT)], zbuf_v)
    pltpu.sync_copy(zbuf_v, out_hbm.at[pl.ds(o0, RPT)])
    pltpu.sync_copy(deg_sh.at[pl.ds(r0, RPT)], dzero_v.at[pl.ds(0, RPT)])
    pltpu.sync_copy(dzero_v.at[pl.ds(0, RPT)], deg_hbm.at[pl.ds(o0, RPT)])


@functools.partial(
    pl.kernel,
    out_type=jax.ShapeDtypeStruct((NC * NPAD,), jnp.float32),
    mesh=_MESH,
    scratch_types=(
        pltpu.VMEM((B,), jnp.int32),
        pltpu.VMEM((B,), jnp.int32),
        pltpu.VMEM((B,), jnp.float32),
        pltpu.VMEM((RPT1,), jnp.float32),
        pltpu.VMEM_SHARED((NPAD,), jnp.float32),
        pltpu.SemaphoreType.DMA,
    ),
)
def _sc_agg_scalar(t_hbm, src_hbm, dst_hbm, out_hbm,
                   src_v, dst_v, vals_v, zbuf_v, acc_sh, sem):
    c = lax.axis_index("c")
    s = lax.axis_index("s")
    epw = E // (NC * NS)
    e0 = (c * NS + s) * epw
    r0 = s * RPT1

    _zero_1d(zbuf_v, RPT1)
    pltpu.sync_copy(zbuf_v, acc_sh.at[pl.ds(r0, RPT1)])
    plsc.subcore_barrier()

    def step(i, carry):
        b0 = e0 + i * B
        pltpu.sync_copy(src_hbm.at[pl.ds(b0, B)], src_v)
        pltpu.sync_copy(dst_hbm.at[pl.ds(b0, B)], dst_v)
        pltpu.async_copy(t_hbm.at[src_v], vals_v, sem).wait()
        pltpu.sync_copy(vals_v, acc_sh.at[dst_v], add=True)
        return carry

    lax.fori_loop(0, epw // B, step, 0)
    plsc.subcore_barrier()

    o0 = c * NPAD + r0
    pltpu.sync_copy(acc_sh.at[pl.ds(r0, RPT1)], zbuf_v)
    pltpu.sync_copy(zbuf_v, out_hbm.at[pl.ds(o0, RPT1)])


def _dense_layer(agg, deg, h, Wn, Ws, b, g, be):
    def body(agg_ref, deg_ref, h_ref, Wn_ref, Ws_ref, b_ref, g_ref, be_ref,
             o_ref):
        deg = jnp.maximum(deg_ref[...], 1.0)
        agg_m = agg_ref[...] / deg
        lin = (jnp.dot(agg_m, Wn_ref[...], preferred_element_type=jnp.float32)
               + jnp.dot(h_ref[...], Ws_ref[...],
                         preferred_element_type=jnp.float32)
               + b_ref[...])
        mu = jnp.mean(lin, axis=0, keepdims=True)
        cen = lin - mu
        var = jnp.mean(cen * cen, axis=0, keepdims=True)
        y = cen * lax.rsqrt(var + EPS) * g_ref[...] + be_ref[...]
        o_ref[...] = jnp.maximum(y, 0.0)

    return pl.pallas_call(
        body,
        out_shape=jax.ShapeDtypeStruct((N, D), jnp.float32),
    )(agg, deg, h, Wn, Ws, b, g, be)


def _proj_layer(h2, W3):
    def body(h_ref, W3_ref, ts_ref):
        ts_ref[...] = jnp.dot(h_ref[...], W3_ref[...],
                              preferred_element_type=jnp.float32)

    return pl.pallas_call(
        body,
        out_shape=jax.ShapeDtypeStruct((N, 2), jnp.float32),
    )(h2, W3)


def _final_layer(agg3p, deg, s, b3):
    def body(agg3p_ref, deg_ref, s_ref, b3_ref, o_ref):
        deg = jnp.maximum(deg_ref[...], 1.0)
        lin = (agg3p_ref[0] + agg3p_ref[1]) / deg + s_ref[...] + b3_ref[...]
        o_ref[...] = jax.nn.sigmoid(lin)

    return pl.pallas_call(
        body,
        out_shape=jax.ShapeDtypeStruct((N, 1), jnp.float32),
    )(agg3p, deg, s, b3)


def _agg_full(table):
    halves = table.reshape(NC, HROWS, D)[:, :HALF]
    return halves.reshape(NC * HALF, D)[:N]


def _deg_full(degh):
    halves = degh.reshape(NC, HROWS)[:, :HALF]
    return halves.reshape(NC * HALF)[:N].reshape(N, 1)


def kernel(x, edge_index, Wn1, Ws1, b1, g1, be1, Wn2, Ws2, b2, g2, be2,
           Wn3, Ws3, b3):
    src = edge_index[0].astype(jnp.int32)
    dst = edge_index[1].astype(jnp.int32)

    srcs_p, dsts_p, cnts_p = _sc_partition(src, dst)

    agg1_raw, deg_raw = _sc_agg(x, srcs_p, dsts_p, cnts_p)
    agg1 = _agg_full(agg1_raw)
    deg = _deg_full(deg_raw)
    h1 = _dense_layer(agg1, deg, x, Wn1, Ws1, b1.reshape(1, D),
                      g1.reshape(1, D), be1.reshape(1, D))

    agg2_raw, _ = _sc_agg(h1, srcs_p, dsts_p, cnts_p)
    agg2 = _agg_full(agg2_raw)
    h2 = _dense_layer(agg2, deg, h1, Wn2, Ws2, b2.reshape(1, D),
                      g2.reshape(1, D), be2.reshape(1, D))

    W3 = jnp.concatenate([Wn3, Ws3], axis=1)
    ts = _proj_layer(h2, W3)
    t = ts[:, 0:1].reshape(N)
    s = ts[:, 1:2]

    agg3p = _sc_agg_scalar(t, src, dst).reshape(NC, NPAD, 1)[:, :N]

    out = _final_layer(agg3p, deg, s, b3.reshape(1, 1))
    return out.reshape(N)


# speedup vs baseline: 5.0736x; 1.1325x over previous
import functools

import jax
import jax.numpy as jnp
from jax import lax
from jax.experimental import pallas as pl
from jax.experimental.pallas import tpu as pltpu
from jax.experimental.pallas import tpu_sc as plsc

N = 10000
E = 320000
D = 128
EPS = 1e-5

NC = 2
NS = 16
EPT = E // NS
B = 80
NB = EPT // B
HALF = 5120
HROWS = 5248
RPT = HROWS // NS

NPAD = 10240
RPT1 = NPAD // NS


def _zero_2d(buf, rows, cols):
    z16 = jnp.zeros((16,), jnp.float32)

    def zr(r, carry):
        for c8 in range(cols // 16):
            buf[r, pl.ds(c8 * 16, 16)] = z16
        return carry

    lax.fori_loop(0, rows, zr, 0)


def _zero_1d(buf, n):
    z16 = jnp.zeros((16,), jnp.float32)

    def zr(r, carry):
        buf[pl.ds(r * 16, 16)] = z16
        return carry

    lax.fori_loop(0, n // 16, zr, 0)


_MESH = plsc.VectorSubcoreMesh(core_axis_name="c", subcore_axis_name="s")

NW = NC * NS
EPW = E // NW
BC = 128
CAP = 10256


@functools.partial(
    pl.kernel,
    out_type=(jax.ShapeDtypeStruct((2 * NW * CAP,), jnp.int32),
              jax.ShapeDtypeStruct((2 * NW * CAP,), jnp.int32),
              jax.ShapeDtypeStruct((2 * NW * 16,), jnp.int32)),
    mesh=_MESH,
    scratch_types=(
        pltpu.VMEM((B,), jnp.int32),
        pltpu.VMEM((B,), jnp.int32),
        pltpu.VMEM((CAP,), jnp.int32),
        pltpu.VMEM((CAP,), jnp.int32),
        pltpu.VMEM((CAP,), jnp.int32),
        pltpu.VMEM((CAP,), jnp.int32),
        pltpu.VMEM((16,), jnp.int32),
        pltpu.SemaphoreType.DMA,
    ),
    compiler_params=pltpu.CompilerParams(needs_layout_passes=False),
)
def _sc_partition(src_hbm, dst_hbm, srcs_out, dsts_out, cnts_out,
                  src_v, dst_v, srcA, dstA, srcB, dstB, cnt_v, sem):
    c = lax.axis_index("c")
    s = lax.axis_index("s")
    w = c * NS + s
    e0 = w * EPW
    lane = lax.iota(jnp.int32, 16)
    trash = jnp.int32(CAP - 16)

    def step(i, cnts):
        ca, cb = cnts
        b0 = e0 + i * B
        pltpu.sync_copy(src_hbm.at[pl.ds(b0, B)], src_v)
        pltpu.sync_copy(dst_hbm.at[pl.ds(b0, B)], dst_v)
        for k in range(B // 16):
            s16 = src_v[pl.ds(k * 16, 16)]
            d16 = dst_v[pl.ds(k * 16, 16)]
            mA = d16 < HALF
            miA = jnp.where(mA, jnp.int32(1), jnp.int32(0))
            posA = jnp.cumsum(miA) - miA
            idxA = jnp.where(mA, ca + posA, trash + lane)
            plsc.store_scatter(srcA, [idxA], s16)
            plsc.store_scatter(dstA, [idxA], d16)
            ca = ca + jnp.sum(miA)
            miB = jnp.int32(1) - miA
            posB = jnp.cumsum(miB) - miB
            idxB = jnp.where(mA, trash + lane, cb + posB)
            plsc.store_scatter(srcB, [idxB], s16)
            plsc.store_scatter(dstB, [idxB], d16 - HALF)
            cb = cb + jnp.sum(miB)
        return ca, cb

    ca, cb = lax.fori_loop(0, EPW // B, step, (jnp.int32(0), jnp.int32(0)))

    zero16 = jnp.zeros((16,), jnp.int32)
    garb16 = jnp.full((16,), HALF, jnp.int32)
    for k in range(BC // 16):
        srcA[pl.ds(ca + k * 16, 16)] = zero16
        dstA[pl.ds(ca + k * 16, 16)] = garb16
        srcB[pl.ds(cb + k * 16, 16)] = zero16
        dstB[pl.ds(cb + k * 16, 16)] = garb16
    ca = ((ca + (BC - 1)) // BC) * BC
    cb = ((cb + (BC - 1)) // BC) * BC

    oA = w * CAP
    oB = (NW + w) * CAP
    pltpu.sync_copy(srcA, srcs_out.at[pl.ds(oA, CAP)])
    pltpu.sync_copy(dstA, dsts_out.at[pl.ds(oA, CAP)])
    pltpu.sync_copy(srcB, srcs_out.at[pl.ds(oB, CAP)])
    pltpu.sync_copy(dstB, dsts_out.at[pl.ds(oB, CAP)])
    cnt_v[pl.ds(0, 16)] = zero16 + ca
    pltpu.sync_copy(cnt_v, cnts_out.at[pl.ds(w * 16, 16)])
    cnt_v[pl.ds(0, 16)] = zero16 + cb
    pltpu.sync_copy(cnt_v, cnts_out.at[pl.ds((NW + w) * 16, 16)])


@functools.partial(
    pl.kernel,
    out_type=(jax.ShapeDtypeStruct((NC * HROWS, D), jnp.float32),
              jax.ShapeDtypeStruct((NC * HROWS,), jnp.float32)),
    mesh=_MESH,
    scratch_types=(
        pltpu.VMEM((BC,), jnp.int32),
        pltpu.VMEM((BC,), jnp.int32),
        pltpu.VMEM((BC, D), jnp.float32),
        pltpu.VMEM((RPT, D), jnp.float32),
        pltpu.VMEM((BC,), jnp.float32),
        pltpu.VMEM((336,), jnp.float32),
        pltpu.VMEM((16,), jnp.int32),
        pltpu.VMEM_SHARED((HROWS, D), jnp.float32),
        pltpu.VMEM_SHARED((HROWS,), jnp.float32),
        pltpu.SemaphoreType.DMA,
    ),
    compiler_params=pltpu.CompilerParams(needs_layout_passes=False),
)
def _sc_agg(h_hbm, srcs_hbm, dsts_hbm, cnts_hbm, out_hbm, deg_hbm,
            src_v, dst_v, rows_v, zbuf_v, ones_v, dzero_v, cnt_v, acc_sh,
            deg_sh, sem):
    c = lax.axis_index("c")
    s = lax.axis_index("s")
    r0 = s * RPT

    _zero_2d(zbuf_v, RPT, D)
    _zero_1d(dzero_v, 336)
    one16 = jnp.full((16,), 1.0, jnp.float32)
    for k in range(BC // 16):
        ones_v[pl.ds(k * 16, 16)] = one16
    pltpu.sync_copy(zbuf_v, acc_sh.at[pl.ds(r0, RPT)])
    pltpu.sync_copy(dzero_v.at[pl.ds(0, RPT)], deg_sh.at[pl.ds(r0, RPT)])
    plsc.subcore_barrier()

    for j in range(2):
        w = s * 2 + j
        region = (c * NW + w) * CAP
        pltpu.sync_copy(cnts_hbm.at[pl.ds((c * NW + w) * 16, 16)], cnt_v)
        cnt = jnp.max(cnt_v[pl.ds(0, 16)])

        def step(i, carry):
            b0 = region + i * BC
            pltpu.sync_copy(srcs_hbm.at[pl.ds(b0, BC)], src_v)
            pltpu.sync_copy(dsts_hbm.at[pl.ds(b0, BC)], dst_v)
            pltpu.async_copy(h_hbm.at[src_v], rows_v, sem).wait()
            pltpu.sync_copy(rows_v, acc_sh.at[dst_v], add=True)
            pltpu.sync_copy(ones_v, deg_sh.at[dst_v], add=True)
            return carry

        lax.fori_loop(0, cnt // BC, step, 0)
    plsc.subcore_barrier()

    o0 = c * HROWS + r0
    pltpu.sync_copy(acc_sh.at[pl.ds(r0, RPT)], zbuf_v)
    pltpu.sync_copy(zbuf_v, out_hbm.at[pl.ds(o0, RPT)])
    pltpu.sync_copy(deg_sh.at[pl.ds(r0, RPT)], dzero_v.at[pl.ds(0, RPT)])
    pltpu.sync_copy(dzero_v.at[pl.ds(0, RPT)], deg_hbm.at[pl.ds(o0, RPT)])


@functools.partial(
    pl.kernel,
    out_type=jax.ShapeDtypeStruct((NW * NPAD,), jnp.float32),
    mesh=_MESH,
    scratch_types=(
        pltpu.VMEM((B,), jnp.int32),
        pltpu.VMEM((B,), jnp.int32),
        pltpu.VMEM((NPAD,), jnp.float32),
        pltpu.VMEM((NPAD,), jnp.float32),
        pltpu.SemaphoreType.DMA,
    ),
    compiler_params=pltpu.CompilerParams(needs_layout_passes=False),
)
def _sc_agg_scalar(t_hbm, src_hbm, dst_hbm, out_hbm,
                   src_v, dst_v, tloc_v, hist_v, sem):
    c = lax.axis_index("c")
    s = lax.axis_index("s")
    w = c * NS + s
    e0 = w * EPW

    _zero_1d(hist_v, NPAD)
    pltpu.sync_copy(t_hbm, tloc_v.at[pl.ds(0, N)])

    def step(i, carry):
        b0 = e0 + i * B
        pltpu.sync_copy(src_hbm.at[pl.ds(b0, B)], src_v)
        pltpu.sync_copy(dst_hbm.at[pl.ds(b0, B)], dst_v)
        for k in range(B // 16):
            s16 = src_v[pl.ds(k * 16, 16)]
            d16 = dst_v[pl.ds(k * 16, 16)]
            vals = plsc.load_gather(tloc_v, [s16])
            plsc.addupdate_scatter(hist_v, [d16], vals)
        return carry

    lax.fori_loop(0, EPW // B, step, 0)
    pltpu.sync_copy(hist_v, out_hbm.at[pl.ds(w * NPAD, NPAD)])


def _dense_layer(agg, deg, h, Wn, Ws, b, g, be):
    def body(agg_ref, deg_ref, h_ref, Wn_ref, Ws_ref, b_ref, g_ref, be_ref,
             o_ref):
        deg = jnp.maximum(deg_ref[...], 1.0)
        agg_m = agg_ref[...] / deg
        lin = (jnp.dot(agg_m, Wn_ref[...], preferred_element_type=jnp.float32)
               + jnp.dot(h_ref[...], Ws_ref[...],
                         preferred_element_type=jnp.float32)
               + b_ref[...])
        mu = jnp.mean(lin, axis=0, keepdims=True)
        cen = lin - mu
        var = jnp.mean(cen * cen, axis=0, keepdims=True)
        y = cen * lax.rsqrt(var + EPS) * g_ref[...] + be_ref[...]
        o_ref[...] = jnp.maximum(y, 0.0)

    return pl.pallas_call(
        body,
        out_shape=jax.ShapeDtypeStruct((N, D), jnp.float32),
    )(agg, deg, h, Wn, Ws, b, g, be)


def _proj_layer(h2, W3):
    def body(h_ref, W3_ref, ts_ref):
        ts_ref[...] = jnp.dot(h_ref[...], W3_ref[...],
                              preferred_element_type=jnp.float32)

    return pl.pallas_call(
        body,
        out_shape=jax.ShapeDtypeStruct((N, 2), jnp.float32),
    )(h2, W3)


def _final_layer(agg3p, deg, s, b3):
    def body(agg3p_ref, deg_ref, s_ref, b3_ref, o_ref):
        deg = jnp.maximum(deg_ref[...], 1.0)
        agg3 = jnp.sum(agg3p_ref[...], axis=1, keepdims=True)
        lin = agg3 / deg + s_ref[...] + b3_ref[...]
        o_ref[...] = jax.nn.sigmoid(lin)

    return pl.pallas_call(
        body,
        out_shape=jax.ShapeDtypeStruct((N, 1), jnp.float32),
    )(agg3p, deg, s, b3)


def _agg_full(table):
    halves = table.reshape(NC, HROWS, D)[:, :HALF]
    return halves.reshape(NC * HALF, D)[:N]


def _deg_full(degh):
    halves = degh.reshape(NC, HROWS)[:, :HALF]
    return halves.reshape(NC * HALF)[:N].reshape(N, 1)


def kernel(x, edge_index, Wn1, Ws1, b1, g1, be1, Wn2, Ws2, b2, g2, be2,
           Wn3, Ws3, b3):
    src = edge_index[0].astype(jnp.int32)
    dst = edge_index[1].astype(jnp.int32)

    srcs_p, dsts_p, cnts_p = _sc_partition(src, dst)

    agg1_raw, deg_raw = _sc_agg(x, srcs_p, dsts_p, cnts_p)
    agg1 = _agg_full(agg1_raw)
    deg = _deg_full(deg_raw)
    h1 = _dense_layer(agg1, deg, x, Wn1, Ws1, b1.reshape(1, D),
                      g1.reshape(1, D), be1.reshape(1, D))

    agg2_raw, _ = _sc_agg(h1, srcs_p, dsts_p, cnts_p)
    agg2 = _agg_full(agg2_raw)
    h2 = _dense_layer(agg2, deg, h1, Wn2, Ws2, b2.reshape(1, D),
                      g2.reshape(1, D), be2.reshape(1, D))

    W3 = jnp.concatenate([Wn3, Ws3], axis=1)
    ts = _proj_layer(h2, W3)
    t = ts[:, 0:1].reshape(N)
    s = ts[:, 1:2]

    agg3p = _sc_agg_scalar(t, src, dst).reshape(NW, NPAD)[:, :N].T

    out = _final_layer(agg3p, deg, s, b3.reshape(1, 1))
    return out.reshape(N)


# speedup vs baseline: 6.7466x; 1.3298x over previous
import functools

import jax
import jax.numpy as jnp
from jax import lax
from jax.experimental import pallas as pl
from jax.experimental.pallas import tpu as pltpu
from jax.experimental.pallas import tpu_sc as plsc

N = 10000
E = 320000
D = 128
EPS = 1e-5

NC = 2
NS = 16
EPT = E // NS
B = 80
NB = EPT // B
HALF = 5120
HROWS = 5248
RPT = HROWS // NS

NPAD = 10240
RPT1 = NPAD // NS


def _zero_2d(buf, rows, cols):
    z16 = jnp.zeros((16,), jnp.float32)

    def zr(r, carry):
        for c8 in range(cols // 16):
            buf[r, pl.ds(c8 * 16, 16)] = z16
        return carry

    lax.fori_loop(0, rows, zr, 0)


def _zero_1d(buf, n):
    z16 = jnp.zeros((16,), jnp.float32)

    def zr(r, carry):
        buf[pl.ds(r * 16, 16)] = z16
        return carry

    lax.fori_loop(0, n // 16, zr, 0)


_MESH = plsc.VectorSubcoreMesh(core_axis_name="c", subcore_axis_name="s")

NW = NC * NS
EPW = E // NW
BC = 128
CAP = 10256


@functools.partial(
    pl.kernel,
    out_type=(jax.ShapeDtypeStruct((2 * NW * CAP,), jnp.int32),
              jax.ShapeDtypeStruct((2 * NW * CAP,), jnp.int32),
              jax.ShapeDtypeStruct((2 * NW * 16,), jnp.int32)),
    mesh=_MESH,
    scratch_types=(
        pltpu.VMEM((EPW,), jnp.int32),
        pltpu.VMEM((EPW,), jnp.int32),
        pltpu.VMEM((CAP,), jnp.int32),
        pltpu.VMEM((CAP,), jnp.int32),
        pltpu.VMEM((CAP,), jnp.int32),
        pltpu.VMEM((CAP,), jnp.int32),
        pltpu.VMEM((16,), jnp.int32),
        pltpu.SemaphoreType.DMA,
    ),
    compiler_params=pltpu.CompilerParams(needs_layout_passes=False),
)
def _sc_partition(src_hbm, dst_hbm, srcs_out, dsts_out, cnts_out,
                  src_v, dst_v, srcA, dstA, srcB, dstB, cnt_v, sem):
    c = lax.axis_index("c")
    s = lax.axis_index("s")
    w = c * NS + s
    e0 = w * EPW
    lane = lax.iota(jnp.int32, 16)
    trash = jnp.int32(CAP - 16)

    pltpu.sync_copy(src_hbm.at[pl.ds(e0, EPW)], src_v)
    pltpu.sync_copy(dst_hbm.at[pl.ds(e0, EPW)], dst_v)

    def step(i, cnts):
        ca, cb = cnts
        j = i * 16
        s16 = src_v[pl.ds(j, 16)]
        d16 = dst_v[pl.ds(j, 16)]
        mA = d16 < HALF
        miA = jnp.where(mA, jnp.int32(1), jnp.int32(0))
        posA = jnp.cumsum(miA) - miA
        idxA = jnp.where(mA, ca + posA, trash + lane)
        plsc.store_scatter(srcA, [idxA], s16)
        plsc.store_scatter(dstA, [idxA], d16)
        ca = ca + jnp.sum(miA)
        miB = jnp.int32(1) - miA
        posB = jnp.cumsum(miB) - miB
        idxB = jnp.where(mA, trash + lane, cb + posB)
        plsc.store_scatter(srcB, [idxB], s16)
        plsc.store_scatter(dstB, [idxB], d16 - HALF)
        cb = cb + jnp.sum(miB)
        return ca, cb

    ca, cb = lax.fori_loop(0, EPW // 16, step, (jnp.int32(0), jnp.int32(0)))

    zero16 = jnp.zeros((16,), jnp.int32)
    garb16 = jnp.full((16,), HALF, jnp.int32)
    for k in range(BC // 16):
        srcA[pl.ds(ca + k * 16, 16)] = zero16
        dstA[pl.ds(ca + k * 16, 16)] = garb16
        srcB[pl.ds(cb + k * 16, 16)] = zero16
        dstB[pl.ds(cb + k * 16, 16)] = garb16
    ca = ((ca + (BC - 1)) // BC) * BC
    cb = ((cb + (BC - 1)) // BC) * BC

    oA = w * CAP
    oB = (NW + w) * CAP
    pltpu.sync_copy(srcA, srcs_out.at[pl.ds(oA, CAP)])
    pltpu.sync_copy(dstA, dsts_out.at[pl.ds(oA, CAP)])
    pltpu.sync_copy(srcB, srcs_out.at[pl.ds(oB, CAP)])
    pltpu.sync_copy(dstB, dsts_out.at[pl.ds(oB, CAP)])
    cnt_v[pl.ds(0, 16)] = zero16 + ca
    pltpu.sync_copy(cnt_v, cnts_out.at[pl.ds(w * 16, 16)])
    cnt_v[pl.ds(0, 16)] = zero16 + cb
    pltpu.sync_copy(cnt_v, cnts_out.at[pl.ds((NW + w) * 16, 16)])


@functools.partial(
    pl.kernel,
    out_type=(jax.ShapeDtypeStruct((NC * HROWS, D), jnp.float32),
              jax.ShapeDtypeStruct((NC * HROWS,), jnp.float32)),
    mesh=_MESH,
    scratch_types=(
        pltpu.VMEM((CAP,), jnp.int32),
        pltpu.VMEM((BC,), jnp.int32),
        pltpu.VMEM((BC, D), jnp.float32),
        pltpu.VMEM((RPT, D), jnp.float32),
        pltpu.VMEM((BC,), jnp.float32),
        pltpu.VMEM((336,), jnp.float32),
        pltpu.VMEM((16,), jnp.int32),
        pltpu.VMEM_SHARED((HROWS, D), jnp.float32),
        pltpu.VMEM_SHARED((HROWS,), jnp.float32),
        pltpu.SemaphoreType.DMA,
    ),
    compiler_params=pltpu.CompilerParams(needs_layout_passes=False),
)
def _sc_agg(h_hbm, srcs_hbm, dsts_hbm, cnts_hbm, out_hbm, deg_hbm,
            src_v, dst_v, rows_v, zbuf_v, ones_v, dzero_v, cnt_v, acc_sh,
            deg_sh, sem):
    c = lax.axis_index("c")
    s = lax.axis_index("s")
    r0 = s * RPT

    _zero_2d(zbuf_v, RPT, D)
    _zero_1d(dzero_v, 336)
    one16 = jnp.full((16,), 1.0, jnp.float32)
    for k in range(BC // 16):
        ones_v[pl.ds(k * 16, 16)] = one16
    pltpu.sync_copy(zbuf_v, acc_sh.at[pl.ds(r0, RPT)])
    pltpu.sync_copy(dzero_v.at[pl.ds(0, RPT)], deg_sh.at[pl.ds(r0, RPT)])
    plsc.subcore_barrier()

    for j in range(2):
        w = s * 2 + j
        region = (c * NW + w) * CAP
        pltpu.sync_copy(cnts_hbm.at[pl.ds((c * NW + w) * 16, 16)], cnt_v)
        cnt = jnp.max(cnt_v[pl.ds(0, 16)])
        pltpu.sync_copy(srcs_hbm.at[pl.ds(region, CAP)], src_v)

        def step(i, carry):
            b0 = region + i * BC
            pltpu.sync_copy(dsts_hbm.at[pl.ds(b0, BC)], dst_v)
            pltpu.async_copy(h_hbm.at[src_v.at[pl.ds(i * BC, BC)]], rows_v,
                             sem).wait()
            pltpu.sync_copy(rows_v, acc_sh.at[dst_v], add=True)
            pltpu.sync_copy(ones_v, deg_sh.at[dst_v], add=True)
            return carry

        lax.fori_loop(0, cnt // BC, step, 0)
    plsc.subcore_barrier()

    o0 = c * HROWS + r0
    pltpu.sync_copy(acc_sh.at[pl.ds(r0, RPT)], zbuf_v)
    pltpu.sync_copy(zbuf_v, out_hbm.at[pl.ds(o0, RPT)])
    pltpu.sync_copy(deg_sh.at[pl.ds(r0, RPT)], dzero_v.at[pl.ds(0, RPT)])
    pltpu.sync_copy(dzero_v.at[pl.ds(0, RPT)], deg_hbm.at[pl.ds(o0, RPT)])


@functools.partial(
    pl.kernel,
    out_type=jax.ShapeDtypeStruct((NW * NPAD,), jnp.float32),
    mesh=_MESH,
    scratch_types=(
        pltpu.VMEM((EPW,), jnp.int32),
        pltpu.VMEM((EPW,), jnp.int32),
        pltpu.VMEM((NPAD,), jnp.float32),
        pltpu.VMEM((NPAD,), jnp.float32),
        pltpu.SemaphoreType.DMA,
    ),
    compiler_params=pltpu.CompilerParams(needs_layout_passes=False),
)
def _sc_agg_scalar(t_hbm, src_hbm, dst_hbm, out_hbm,
                   src_v, dst_v, tloc_v, hist_v, sem):
    c = lax.axis_index("c")
    s = lax.axis_index("s")
    w = c * NS + s
    e0 = w * EPW

    _zero_1d(hist_v, NPAD)
    pltpu.sync_copy(t_hbm, tloc_v.at[pl.ds(0, N)])
    pltpu.sync_copy(src_hbm.at[pl.ds(e0, EPW)], src_v)
    pltpu.sync_copy(dst_hbm.at[pl.ds(e0, EPW)], dst_v)

    def step(i, carry):
        j = i * 16
        s16 = src_v[pl.ds(j, 16)]
        d16 = dst_v[pl.ds(j, 16)]
        vals = plsc.load_gather(tloc_v, [s16])
        plsc.addupdate_scatter(hist_v, [d16], vals)
        return carry

    lax.fori_loop(0, EPW // 16, step, 0)
    pltpu.sync_copy(hist_v, out_hbm.at[pl.ds(w * NPAD, NPAD)])


def _dense_layer(agg, deg, h, Wn, Ws, b, g, be):
    def body(agg_ref, deg_ref, h_ref, Wn_ref, Ws_ref, b_ref, g_ref, be_ref,
             o_ref):
        deg = jnp.maximum(deg_ref[...], 1.0)
        agg_m = agg_ref[...] / deg
        lin = (jnp.dot(agg_m, Wn_ref[...], preferred_element_type=jnp.float32)
               + jnp.dot(h_ref[...], Ws_ref[...],
                         preferred_element_type=jnp.float32)
               + b_ref[...])
        mu = jnp.mean(lin, axis=0, keepdims=True)
        cen = lin - mu
        var = jnp.mean(cen * cen, axis=0, keepdims=True)
        y = cen * lax.rsqrt(var + EPS) * g_ref[...] + be_ref[...]
        o_ref[...] = jnp.maximum(y, 0.0)

    return pl.pallas_call(
        body,
        out_shape=jax.ShapeDtypeStruct((N, D), jnp.float32),
    )(agg, deg, h, Wn, Ws, b, g, be)


def _proj_layer(h2, W3):
    def body(h_ref, W3_ref, ts_ref):
        ts_ref[...] = jnp.dot(h_ref[...], W3_ref[...],
                              preferred_element_type=jnp.float32)

    return pl.pallas_call(
        body,
        out_shape=jax.ShapeDtypeStruct((N, 2), jnp.float32),
    )(h2, W3)


def _final_layer(agg3p, deg, s, b3):
    def body(agg3p_ref, deg_ref, s_ref, b3_ref, o_ref):
        deg = jnp.maximum(deg_ref[...], 1.0)
        agg3 = jnp.sum(agg3p_ref[...], axis=1, keepdims=True)
        lin = agg3 / deg + s_ref[...] + b3_ref[...]
        o_ref[...] = jax.nn.sigmoid(lin)

    return pl.pallas_call(
        body,
        out_shape=jax.ShapeDtypeStruct((N, 1), jnp.float32),
    )(agg3p, deg, s, b3)


def _agg_full(table):
    halves = table.reshape(NC, HROWS, D)[:, :HALF]
    return halves.reshape(NC * HALF, D)[:N]


def _deg_full(degh):
    halves = degh.reshape(NC, HROWS)[:, :HALF]
    return halves.reshape(NC * HALF)[:N].reshape(N, 1)


def kernel(x, edge_index, Wn1, Ws1, b1, g1, be1, Wn2, Ws2, b2, g2, be2,
           Wn3, Ws3, b3):
    src = edge_index[0].astype(jnp.int32)
    dst = edge_index[1].astype(jnp.int32)

    srcs_p, dsts_p, cnts_p = _sc_partition(src, dst)

    agg1_raw, deg_raw = _sc_agg(x, srcs_p, dsts_p, cnts_p)
    agg1 = _agg_full(agg1_raw)
    deg = _deg_full(deg_raw)
    h1 = _dense_layer(agg1, deg, x, Wn1, Ws1, b1.reshape(1, D),
                      g1.reshape(1, D), be1.reshape(1, D))

    agg2_raw, _ = _sc_agg(h1, srcs_p, dsts_p, cnts_p)
    agg2 = _agg_full(agg2_raw)
    h2 = _dense_layer(agg2, deg, h1, Wn2, Ws2, b2.reshape(1, D),
                      g2.reshape(1, D), be2.reshape(1, D))

    W3 = jnp.concatenate([Wn3, Ws3], axis=1)
    ts = _proj_layer(h2, W3)
    t = ts[:, 0:1].reshape(N)
    s = ts[:, 1:2]

    agg3p = _sc_agg_scalar(t, src, dst).reshape(NW, NPAD)[:, :N].T

    out = _final_layer(agg3p, deg, s, b3.reshape(1, 1))
    return out.reshape(N)


# speedup vs baseline: 7.6161x; 1.1289x over previous
import functools

import jax
import jax.numpy as jnp
from jax import lax
from jax.experimental import pallas as pl
from jax.experimental.pallas import tpu as pltpu
from jax.experimental.pallas import tpu_sc as plsc

N = 10000
E = 320000
D = 128
EPS = 1e-5

NC = 2
NS = 16
EPT = E // NS
B = 80
NB = EPT // B
HALF = 5120
HROWS = 5248
RPT = HROWS // NS

NPAD = 10240
RPT1 = NPAD // NS


def _zero_2d(buf, rows, cols):
    z16 = jnp.zeros((16,), jnp.float32)

    def zr(r, carry):
        for c8 in range(cols // 16):
            buf[r, pl.ds(c8 * 16, 16)] = z16
        return carry

    lax.fori_loop(0, rows, zr, 0)


def _zero_1d(buf, n):
    z16 = jnp.zeros((16,), jnp.float32)

    def zr(r, carry):
        buf[pl.ds(r * 16, 16)] = z16
        return carry

    lax.fori_loop(0, n // 16, zr, 0)


_MESH = plsc.VectorSubcoreMesh(core_axis_name="c", subcore_axis_name="s")

NW = NC * NS
EPW = E // NW
BC = 128
CAP = 10256


@functools.partial(
    pl.kernel,
    out_type=(jax.ShapeDtypeStruct((2 * NW * CAP,), jnp.int32),
              jax.ShapeDtypeStruct((2 * NW * CAP,), jnp.int32),
              jax.ShapeDtypeStruct((2 * NW * 16,), jnp.int32)),
    mesh=_MESH,
    scratch_types=(
        pltpu.VMEM((EPW,), jnp.int32),
        pltpu.VMEM((EPW,), jnp.int32),
        pltpu.VMEM((CAP,), jnp.int32),
        pltpu.VMEM((CAP,), jnp.int32),
        pltpu.VMEM((CAP,), jnp.int32),
        pltpu.VMEM((CAP,), jnp.int32),
        pltpu.VMEM((16,), jnp.int32),
        pltpu.SemaphoreType.DMA,
    ),
    compiler_params=pltpu.CompilerParams(needs_layout_passes=False),
)
def _sc_partition(src_hbm, dst_hbm, srcs_out, dsts_out, cnts_out,
                  src_v, dst_v, srcA, dstA, srcB, dstB, cnt_v, sem):
    c = lax.axis_index("c")
    s = lax.axis_index("s")
    w = c * NS + s
    e0 = w * EPW
    lane = lax.iota(jnp.int32, 16)
    trash = jnp.int32(CAP - 16)

    pltpu.sync_copy(src_hbm.at[pl.ds(e0, EPW)], src_v)
    pltpu.sync_copy(dst_hbm.at[pl.ds(e0, EPW)], dst_v)

    def step(i, cnts):
        ca, cb = cnts
        j = i * 16
        s16 = src_v[pl.ds(j, 16)]
        d16 = dst_v[pl.ds(j, 16)]
        mA = d16 < HALF
        miA = jnp.where(mA, jnp.int32(1), jnp.int32(0))
        posA = jnp.cumsum(miA) - miA
        idxA = jnp.where(mA, ca + posA, trash + lane)
        plsc.store_scatter(srcA, [idxA], s16)
        plsc.store_scatter(dstA, [idxA], d16)
        ca = ca + jnp.sum(miA)
        miB = jnp.int32(1) - miA
        posB = jnp.cumsum(miB) - miB
        idxB = jnp.where(mA, trash + lane, cb + posB)
        plsc.store_scatter(srcB, [idxB], s16)
        plsc.store_scatter(dstB, [idxB], d16 - HALF)
        cb = cb + jnp.sum(miB)
        return ca, cb

    ca, cb = lax.fori_loop(0, EPW // 16, step, (jnp.int32(0), jnp.int32(0)))

    zero16 = jnp.zeros((16,), jnp.int32)
    garb16 = jnp.full((16,), HALF, jnp.int32)
    for k in range(BC // 16):
        srcA[pl.ds(ca + k * 16, 16)] = zero16
        dstA[pl.ds(ca + k * 16, 16)] = garb16
        srcB[pl.ds(cb + k * 16, 16)] = zero16
        dstB[pl.ds(cb + k * 16, 16)] = garb16
    ca = ((ca + (BC - 1)) // BC) * BC
    cb = ((cb + (BC - 1)) // BC) * BC

    oA = w * CAP
    oB = (NW + w) * CAP
    pltpu.sync_copy(srcA, srcs_out.at[pl.ds(oA, CAP)])
    pltpu.sync_copy(dstA, dsts_out.at[pl.ds(oA, CAP)])
    pltpu.sync_copy(srcB, srcs_out.at[pl.ds(oB, CAP)])
    pltpu.sync_copy(dstB, dsts_out.at[pl.ds(oB, CAP)])
    cnt_v[pl.ds(0, 16)] = zero16 + ca
    pltpu.sync_copy(cnt_v, cnts_out.at[pl.ds(w * 16, 16)])
    cnt_v[pl.ds(0, 16)] = zero16 + cb
    pltpu.sync_copy(cnt_v, cnts_out.at[pl.ds((NW + w) * 16, 16)])


@functools.partial(
    pl.kernel,
    out_type=(jax.ShapeDtypeStruct((NC * HROWS, D), jnp.float32),
              jax.ShapeDtypeStruct((NC * HROWS,), jnp.float32)),
    mesh=_MESH,
    scratch_types=(
        pltpu.VMEM((CAP,), jnp.int32),
        pltpu.VMEM((BC,), jnp.int32),
        pltpu.VMEM((BC,), jnp.int32),
        pltpu.VMEM((BC, D), jnp.float32),
        pltpu.VMEM((BC, D), jnp.float32),
        pltpu.VMEM((RPT, D), jnp.float32),
        pltpu.VMEM((BC,), jnp.float32),
        pltpu.VMEM((336,), jnp.float32),
        pltpu.VMEM((16,), jnp.int32),
        pltpu.VMEM_SHARED((HROWS, D), jnp.float32),
        pltpu.VMEM_SHARED((HROWS,), jnp.float32),
        pltpu.SemaphoreType.DMA,
        pltpu.SemaphoreType.DMA,
        pltpu.SemaphoreType.DMA,
        pltpu.SemaphoreType.DMA,
        pltpu.SemaphoreType.DMA,
    ),
    compiler_params=pltpu.CompilerParams(needs_layout_passes=False),
)
def _sc_agg(h_hbm, srcs_hbm, dsts_hbm, cnts_hbm, out_hbm, deg_hbm,
            src_v, dst0_v, dst1_v, rows0_v, rows1_v, zbuf_v, ones_v, dzero_v,
            cnt_v, acc_sh, deg_sh, sem, semS0, semS1, semD0, semD1):
    c = lax.axis_index("c")
    s = lax.axis_index("s")
    r0 = s * RPT

    _zero_2d(zbuf_v, RPT, D)
    _zero_1d(dzero_v, 336)
    one16 = jnp.full((16,), 1.0, jnp.float32)
    for k in range(BC // 16):
        ones_v[pl.ds(k * 16, 16)] = one16
    pltpu.sync_copy(zbuf_v, acc_sh.at[pl.ds(r0, RPT)])
    pltpu.sync_copy(dzero_v.at[pl.ds(0, RPT)], deg_sh.at[pl.ds(r0, RPT)])
    plsc.subcore_barrier()

    for j in range(2):
        w = s * 2 + j
        region = (c * NW + w) * CAP
        pltpu.sync_copy(cnts_hbm.at[pl.ds((c * NW + w) * 16, 16)], cnt_v)
        cnt = jnp.max(cnt_v[pl.ds(0, 16)])
        pltpu.sync_copy(srcs_hbm.at[pl.ds(region, CAP)], src_v)
        nb = cnt // BC
        npairs = nb // 2
        odd = nb - 2 * npairs

        def halfstep(i, dst_v, rows_v, semS, semD, first):
            @pl.when(jnp.logical_not(first))
            def _():
                pltpu.make_async_copy(rows_v, acc_sh.at[dst_v], semS).wait()
                pltpu.make_async_copy(ones_v, deg_sh.at[dst_v], semD).wait()
            b0 = region + i * BC
            pltpu.sync_copy(dsts_hbm.at[pl.ds(b0, BC)], dst_v)
            pltpu.async_copy(h_hbm.at[src_v.at[pl.ds(i * BC, BC)]], rows_v,
                             sem).wait()
            pltpu.async_copy(rows_v, acc_sh.at[dst_v], semS, add=True)
            pltpu.async_copy(ones_v, deg_sh.at[dst_v], semD, add=True)

        def pair(p, carry):
            halfstep(2 * p, dst0_v, rows0_v, semS0, semD0, p == 0)
            halfstep(2 * p + 1, dst1_v, rows1_v, semS1, semD1, p == 0)
            return carry

        lax.fori_loop(0, npairs, pair, 0)

        @pl.when(npairs > 0)
        def _():
            pltpu.make_async_copy(rows0_v, acc_sh.at[dst0_v], semS0).wait()
            pltpu.make_async_copy(ones_v, deg_sh.at[dst0_v], semD0).wait()
            pltpu.make_async_copy(rows1_v, acc_sh.at[dst1_v], semS1).wait()
            pltpu.make_async_copy(ones_v, deg_sh.at[dst1_v], semD1).wait()

        @pl.when(odd > 0)
        def _():
            b0 = region + 2 * npairs * BC
            pltpu.sync_copy(dsts_hbm.at[pl.ds(b0, BC)], dst0_v)
            pltpu.async_copy(h_hbm.at[src_v.at[pl.ds(2 * npairs * BC, BC)]],
                             rows0_v, sem).wait()
            pltpu.sync_copy(rows0_v, acc_sh.at[dst0_v], add=True)
            pltpu.sync_copy(ones_v, deg_sh.at[dst0_v], add=True)
    plsc.subcore_barrier()

    o0 = c * HROWS + r0
    pltpu.sync_copy(acc_sh.at[pl.ds(r0, RPT)], zbuf_v)
    pltpu.sync_copy(zbuf_v, out_hbm.at[pl.ds(o0, RPT)])
    pltpu.sync_copy(deg_sh.at[pl.ds(r0, RPT)], dzero_v.at[pl.ds(0, RPT)])
    pltpu.sync_copy(dzero_v.at[pl.ds(0, RPT)], deg_hbm.at[pl.ds(o0, RPT)])


@functools.partial(
    pl.kernel,
    out_type=jax.ShapeDtypeStruct((NW * NPAD,), jnp.float32),
    mesh=_MESH,
    scratch_types=(
        pltpu.VMEM((EPW,), jnp.int32),
        pltpu.VMEM((EPW,), jnp.int32),
        pltpu.VMEM((NPAD,), jnp.float32),
        pltpu.VMEM((NPAD,), jnp.float32),
        pltpu.SemaphoreType.DMA,
    ),
    compiler_params=pltpu.CompilerParams(needs_layout_passes=False),
)
def _sc_agg_scalar(t_hbm, src_hbm, dst_hbm, out_hbm,
                   src_v, dst_v, tloc_v, hist_v, sem):
    c = lax.axis_index("c")
    s = lax.axis_index("s")
    w = c * NS + s
    e0 = w * EPW

    _zero_1d(hist_v, NPAD)
    pltpu.sync_copy(t_hbm, tloc_v.at[pl.ds(0, N)])
    pltpu.sync_copy(src_hbm.at[pl.ds(e0, EPW)], src_v)
    pltpu.sync_copy(dst_hbm.at[pl.ds(e0, EPW)], dst_v)

    def step(i, carry):
        j = i * 16
        s16 = src_v[pl.ds(j, 16)]
        d16 = dst_v[pl.ds(j, 16)]
        vals = plsc.load_gather(tloc_v, [s16])
        plsc.addupdate_scatter(hist_v, [d16], vals)
        return carry

    lax.fori_loop(0, EPW // 16, step, 0)
    pltpu.sync_copy(hist_v, out_hbm.at[pl.ds(w * NPAD, NPAD)])


def _dense_layer(agg, deg, h, Wn, Ws, b, g, be):
    def body(agg_ref, deg_ref, h_ref, Wn_ref, Ws_ref, b_ref, g_ref, be_ref,
             o_ref):
        deg = jnp.maximum(deg_ref[...], 1.0)
        agg_m = agg_ref[...] / deg
        lin = (jnp.dot(agg_m, Wn_ref[...], preferred_element_type=jnp.float32)
               + jnp.dot(h_ref[...], Ws_ref[...],
                         preferred_element_type=jnp.float32)
               + b_ref[...])
        mu = jnp.mean(lin, axis=0, keepdims=True)
        cen = lin - mu
        var = jnp.mean(cen * cen, axis=0, keepdims=True)
        y = cen * lax.rsqrt(var + EPS) * g_ref[...] + be_ref[...]
        o_ref[...] = jnp.maximum(y, 0.0)

    return pl.pallas_call(
        body,
        out_shape=jax.ShapeDtypeStruct((N, D), jnp.float32),
    )(agg, deg, h, Wn, Ws, b, g, be)


def _proj_layer(h2, W3):
    def body(h_ref, W3_ref, ts_ref):
        ts_ref[...] = jnp.dot(h_ref[...], W3_ref[...],
                              preferred_element_type=jnp.float32)

    return pl.pallas_call(
        body,
        out_shape=jax.ShapeDtypeStruct((N, 2), jnp.float32),
    )(h2, W3)


def _final_layer(agg3p, deg, s, b3):
    def body(agg3p_ref, deg_ref, s_ref, b3_ref, o_ref):
        deg = jnp.maximum(deg_ref[...], 1.0)
        agg3 = jnp.sum(agg3p_ref[...], axis=1, keepdims=True)
        lin = agg3 / deg + s_ref[...] + b3_ref[...]
        o_ref[...] = jax.nn.sigmoid(lin)

    return pl.pallas_call(
        body,
        out_shape=jax.ShapeDtypeStruct((N, 1), jnp.float32),
    )(agg3p, deg, s, b3)


def _agg_full(table):
    halves = table.reshape(NC, HROWS, D)[:, :HALF]
    return halves.reshape(NC * HALF, D)[:N]


def _deg_full(degh):
    halves = degh.reshape(NC, HROWS)[:, :HALF]
    return halves.reshape(NC * HALF)[:N].reshape(N, 1)


def kernel(x, edge_index, Wn1, Ws1, b1, g1, be1, Wn2, Ws2, b2, g2, be2,
           Wn3, Ws3, b3):
    src = edge_index[0].astype(jnp.int32)
    dst = edge_index[1].astype(jnp.int32)

    srcs_p, dsts_p, cnts_p = _sc_partition(src, dst)

    agg1_raw, deg_raw = _sc_agg(x, srcs_p, dsts_p, cnts_p)
    agg1 = _agg_full(agg1_raw)
    deg = _deg_full(deg_raw)
    h1 = _dense_layer(agg1, deg, x, Wn1, Ws1, b1.reshape(1, D),
                      g1.reshape(1, D), be1.reshape(1, D))

    agg2_raw, _ = _sc_agg(h1, srcs_p, dsts_p, cnts_p)
    agg2 = _agg_full(agg2_raw)
    h2 = _dense_layer(agg2, deg, h1, Wn2, Ws2, b2.reshape(1, D),
                      g2.reshape(1, D), be2.reshape(1, D))

    W3 = jnp.concatenate([Wn3, Ws3], axis=1)
    ts = _proj_layer(h2, W3)
    t = ts[:, 0:1].reshape(N)
    s = ts[:, 1:2]

    agg3p = _sc_agg_scalar(t, src, dst).reshape(NW, NPAD)[:, :N].T

    out = _final_layer(agg3p, deg, s, b3.reshape(1, 1))
    return out.reshape(N)


# speedup vs baseline: 7.6979x; 1.0107x over previous
import functools

import jax
import jax.numpy as jnp
from jax import lax
from jax.experimental import pallas as pl
from jax.experimental.pallas import tpu as pltpu
from jax.experimental.pallas import tpu_sc as plsc

N = 10000
E = 320000
D = 128
EPS = 1e-5

NC = 2
NS = 16
EPT = E // NS
B = 80
NB = EPT // B
HALF = 5120
HROWS = 5248
RPT = HROWS // NS

NPAD = 10240
RPT1 = NPAD // NS


def _zero_2d(buf, rows, cols):
    z16 = jnp.zeros((16,), jnp.float32)

    def zr(r, carry):
        for c8 in range(cols // 16):
            buf[r, pl.ds(c8 * 16, 16)] = z16
        return carry

    lax.fori_loop(0, rows, zr, 0)


def _zero_1d(buf, n):
    z16 = jnp.zeros((16,), jnp.float32)

    def zr(r, carry):
        buf[pl.ds(r * 16, 16)] = z16
        return carry

    lax.fori_loop(0, n // 16, zr, 0)


_MESH = plsc.VectorSubcoreMesh(core_axis_name="c", subcore_axis_name="s")

NW = NC * NS
EPW = E // NW
BC = 128
CAP = 10256


@functools.partial(
    pl.kernel,
    out_type=(jax.ShapeDtypeStruct((2 * NW * CAP,), jnp.int32),
              jax.ShapeDtypeStruct((2 * NW * CAP,), jnp.int32),
              jax.ShapeDtypeStruct((2 * NW * 16,), jnp.int32)),
    mesh=_MESH,
    scratch_types=(
        pltpu.VMEM((EPW,), jnp.int32),
        pltpu.VMEM((EPW,), jnp.int32),
        pltpu.VMEM((CAP,), jnp.int32),
        pltpu.VMEM((CAP,), jnp.int32),
        pltpu.VMEM((CAP,), jnp.int32),
        pltpu.VMEM((CAP,), jnp.int32),
        pltpu.VMEM((16,), jnp.int32),
        pltpu.SemaphoreType.DMA,
    ),
    compiler_params=pltpu.CompilerParams(needs_layout_passes=False),
)
def _sc_partition(src_hbm, dst_hbm, srcs_out, dsts_out, cnts_out,
                  src_v, dst_v, srcA, dstA, srcB, dstB, cnt_v, sem):
    c = lax.axis_index("c")
    s = lax.axis_index("s")
    w = c * NS + s
    e0 = w * EPW
    lane = lax.iota(jnp.int32, 16)
    trash = jnp.int32(CAP - 16)

    pltpu.sync_copy(src_hbm.at[pl.ds(e0, EPW)], src_v)
    pltpu.sync_copy(dst_hbm.at[pl.ds(e0, EPW)], dst_v)

    def step(i, cnts):
        ca, cb = cnts
        j = i * 16
        s16 = src_v[pl.ds(j, 16)]
        d16 = dst_v[pl.ds(j, 16)]
        mA = d16 < HALF
        miA = jnp.where(mA, jnp.int32(1), jnp.int32(0))
        posA = jnp.cumsum(miA) - miA
        idxA = jnp.where(mA, ca + posA, trash + lane)
        plsc.store_scatter(srcA, [idxA], s16)
        plsc.store_scatter(dstA, [idxA], d16)
        ca = ca + jnp.sum(miA)
        miB = jnp.int32(1) - miA
        posB = jnp.cumsum(miB) - miB
        idxB = jnp.where(mA, trash + lane, cb + posB)
        plsc.store_scatter(srcB, [idxB], s16)
        plsc.store_scatter(dstB, [idxB], d16 - HALF)
        cb = cb + jnp.sum(miB)
        return ca, cb

    ca, cb = lax.fori_loop(0, EPW // 16, step, (jnp.int32(0), jnp.int32(0)))

    zero16 = jnp.zeros((16,), jnp.int32)
    garb16 = jnp.full((16,), HALF, jnp.int32)
    for k in range(BC // 16):
        srcA[pl.ds(ca + k * 16, 16)] = zero16
        dstA[pl.ds(ca + k * 16, 16)] = garb16
        srcB[pl.ds(cb + k * 16, 16)] = zero16
        dstB[pl.ds(cb + k * 16, 16)] = garb16
    ca = ((ca + (BC - 1)) // BC) * BC
    cb = ((cb + (BC - 1)) // BC) * BC

    oA = w * CAP
    oB = (NW + w) * CAP
    pltpu.sync_copy(srcA, srcs_out.at[pl.ds(oA, CAP)])
    pltpu.sync_copy(dstA, dsts_out.at[pl.ds(oA, CAP)])
    pltpu.sync_copy(srcB, srcs_out.at[pl.ds(oB, CAP)])
    pltpu.sync_copy(dstB, dsts_out.at[pl.ds(oB, CAP)])
    cnt_v[pl.ds(0, 16)] = zero16 + ca
    pltpu.sync_copy(cnt_v, cnts_out.at[pl.ds(w * 16, 16)])
    cnt_v[pl.ds(0, 16)] = zero16 + cb
    pltpu.sync_copy(cnt_v, cnts_out.at[pl.ds((NW + w) * 16, 16)])


def _make_sc_agg(with_deg):
    out_type = [jax.ShapeDtypeStruct((NC * HROWS, D), jnp.float32)]
    scratch = [
        pltpu.VMEM((CAP,), jnp.int32),
        pltpu.VMEM((BC,), jnp.int32),
        pltpu.VMEM((BC,), jnp.int32),
        pltpu.VMEM((BC, D), jnp.float32),
        pltpu.VMEM((BC, D), jnp.float32),
        pltpu.VMEM((RPT, D), jnp.float32),
        pltpu.VMEM((16,), jnp.int32),
        pltpu.VMEM_SHARED((HROWS, D), jnp.float32),
        pltpu.SemaphoreType.DMA,
        pltpu.SemaphoreType.DMA,
        pltpu.SemaphoreType.DMA,
    ]
    if with_deg:
        out_type.append(jax.ShapeDtypeStruct((NC * HROWS,), jnp.float32))
        scratch += [
            pltpu.VMEM((BC,), jnp.float32),
            pltpu.VMEM((336,), jnp.float32),
            pltpu.VMEM_SHARED((HROWS,), jnp.float32),
            pltpu.SemaphoreType.DMA,
            pltpu.SemaphoreType.DMA,
        ]

    @functools.partial(
        pl.kernel,
        out_type=tuple(out_type) if with_deg else out_type[0],
        mesh=_MESH,
        scratch_types=tuple(scratch),
        compiler_params=pltpu.CompilerParams(needs_layout_passes=False),
    )
    def agg(*refs):
        if with_deg:
            (h_hbm, srcs_hbm, dsts_hbm, cnts_hbm, out_hbm, deg_hbm,
             src_v, dst0_v, dst1_v, rows0_v, rows1_v, zbuf_v, cnt_v, acc_sh,
             sem, semS0, semS1,
             ones_v, dzero_v, deg_sh, semD0, semD1) = refs
        else:
            (h_hbm, srcs_hbm, dsts_hbm, cnts_hbm, out_hbm,
             src_v, dst0_v, dst1_v, rows0_v, rows1_v, zbuf_v, cnt_v, acc_sh,
             sem, semS0, semS1) = refs
        c = lax.axis_index("c")
        s = lax.axis_index("s")
        r0 = s * RPT

        _zero_2d(zbuf_v, RPT, D)
        pltpu.sync_copy(zbuf_v, acc_sh.at[pl.ds(r0, RPT)])
        if with_deg:
            _zero_1d(dzero_v, 336)
            one16 = jnp.full((16,), 1.0, jnp.float32)
            for k in range(BC // 16):
                ones_v[pl.ds(k * 16, 16)] = one16
            pltpu.sync_copy(dzero_v.at[pl.ds(0, RPT)],
                            deg_sh.at[pl.ds(r0, RPT)])
        plsc.subcore_barrier()

        for j in range(2):
            w = s * 2 + j
            region = (c * NW + w) * CAP
            pltpu.sync_copy(cnts_hbm.at[pl.ds((c * NW + w) * 16, 16)], cnt_v)
            cnt = jnp.max(cnt_v[pl.ds(0, 16)])
            pltpu.sync_copy(srcs_hbm.at[pl.ds(region, CAP)], src_v)
            nb = cnt // BC
            npairs = nb // 2
            odd = nb - 2 * npairs

            def halfstep(i, dst_v, rows_v, semS, semD, first):
                @pl.when(jnp.logical_not(first))
                def _():
                    pltpu.make_async_copy(rows_v, acc_sh.at[dst_v],
                                          semS).wait()
                    if with_deg:
                        pltpu.make_async_copy(ones_v, deg_sh.at[dst_v],
                                              semD).wait()
                b0 = region + i * BC
                pltpu.sync_copy(dsts_hbm.at[pl.ds(b0, BC)], dst_v)
                pltpu.async_copy(h_hbm.at[src_v.at[pl.ds(i * BC, BC)]],
                                 rows_v, sem).wait()
                pltpu.async_copy(rows_v, acc_sh.at[dst_v], semS, add=True)
                if with_deg:
                    pltpu.async_copy(ones_v, deg_sh.at[dst_v], semD, add=True)

            def pair(p, carry):
                halfstep(2 * p, dst0_v, rows0_v, semS0,
                         semD0 if with_deg else None, p == 0)
                halfstep(2 * p + 1, dst1_v, rows1_v, semS1,
                         semD1 if with_deg else None, p == 0)
                return carry

            lax.fori_loop(0, npairs, pair, 0)

            @pl.when(npairs > 0)
            def _():
                pltpu.make_async_copy(rows0_v, acc_sh.at[dst0_v],
                                      semS0).wait()
                pltpu.make_async_copy(rows1_v, acc_sh.at[dst1_v],
                                      semS1).wait()
                if with_deg:
                    pltpu.make_async_copy(ones_v, deg_sh.at[dst0_v],
                                          semD0).wait()
                    pltpu.make_async_copy(ones_v, deg_sh.at[dst1_v],
                                          semD1).wait()

            @pl.when(odd > 0)
            def _():
                b0 = region + 2 * npairs * BC
                pltpu.sync_copy(dsts_hbm.at[pl.ds(b0, BC)], dst0_v)
                pltpu.async_copy(
                    h_hbm.at[src_v.at[pl.ds(2 * npairs * BC, BC)]],
                    rows0_v, sem).wait()
                pltpu.sync_copy(rows0_v, acc_sh.at[dst0_v], add=True)
                if with_deg:
                    pltpu.sync_copy(ones_v, deg_sh.at[dst0_v], add=True)
        plsc.subcore_barrier()

        o0 = c * HROWS + r0
        pltpu.sync_copy(acc_sh.at[pl.ds(r0, RPT)], zbuf_v)
        pltpu.sync_copy(zbuf_v, out_hbm.at[pl.ds(o0, RPT)])
        if with_deg:
            pltpu.sync_copy(deg_sh.at[pl.ds(r0, RPT)],
                            dzero_v.at[pl.ds(0, RPT)])
            pltpu.sync_copy(dzero_v.at[pl.ds(0, RPT)],
                            deg_hbm.at[pl.ds(o0, RPT)])

    return agg


_sc_agg_deg = _make_sc_agg(True)
_sc_agg_nodeg = _make_sc_agg(False)


@functools.partial(
    pl.kernel,
    out_type=jax.ShapeDtypeStruct((NW * NPAD,), jnp.float32),
    mesh=_MESH,
    scratch_types=(
        pltpu.VMEM((EPW,), jnp.int32),
        pltpu.VMEM((EPW,), jnp.int32),
        pltpu.VMEM((NPAD,), jnp.float32),
        pltpu.VMEM((NPAD,), jnp.float32),
        pltpu.SemaphoreType.DMA,
    ),
    compiler_params=pltpu.CompilerParams(needs_layout_passes=False),
)
def _sc_agg_scalar(t_hbm, src_hbm, dst_hbm, out_hbm,
                   src_v, dst_v, tloc_v, hist_v, sem):
    c = lax.axis_index("c")
    s = lax.axis_index("s")
    w = c * NS + s
    e0 = w * EPW

    _zero_1d(hist_v, NPAD)
    pltpu.sync_copy(t_hbm, tloc_v.at[pl.ds(0, N)])
    pltpu.sync_copy(src_hbm.at[pl.ds(e0, EPW)], src_v)
    pltpu.sync_copy(dst_hbm.at[pl.ds(e0, EPW)], dst_v)

    def step(i, carry):
        j = i * 16
        s16 = src_v[pl.ds(j, 16)]
        d16 = dst_v[pl.ds(j, 16)]
        vals = plsc.load_gather(tloc_v, [s16])
        plsc.addupdate_scatter(hist_v, [d16], vals)
        return carry

    lax.fori_loop(0, EPW // 16, step, 0)
    pltpu.sync_copy(hist_v, out_hbm.at[pl.ds(w * NPAD, NPAD)])


def _dense_layer(agg, deg, h, Wn, Ws, b, g, be):
    def body(agg_ref, deg_ref, h_ref, Wn_ref, Ws_ref, b_ref, g_ref, be_ref,
             o_ref):
        deg = jnp.maximum(deg_ref[...], 1.0)
        agg_m = agg_ref[...] / deg
        lin = (jnp.dot(agg_m, Wn_ref[...], preferred_element_type=jnp.float32)
               + jnp.dot(h_ref[...], Ws_ref[...],
                         preferred_element_type=jnp.float32)
               + b_ref[...])
        mu = jnp.mean(lin, axis=0, keepdims=True)
        cen = lin - mu
        var = jnp.mean(cen * cen, axis=0, keepdims=True)
        y = cen * lax.rsqrt(var + EPS) * g_ref[...] + be_ref[...]
        o_ref[...] = jnp.maximum(y, 0.0)

    return pl.pallas_call(
        body,
        out_shape=jax.ShapeDtypeStruct((N, D), jnp.float32),
    )(agg, deg, h, Wn, Ws, b, g, be)


def _dense_layer2(agg, deg, h, Wn, Ws, b, g, be, W3):
    def body(agg_ref, deg_ref, h_ref, Wn_ref, Ws_ref, b_ref, g_ref, be_ref,
             W3_ref, ts_ref):
        deg = jnp.maximum(deg_ref[...], 1.0)
        agg_m = agg_ref[...] / deg
        lin = (jnp.dot(agg_m, Wn_ref[...], preferred_element_type=jnp.float32)
               + jnp.dot(h_ref[...], Ws_ref[...],
                         preferred_element_type=jnp.float32)
               + b_ref[...])
        mu = jnp.mean(lin, axis=0, keepdims=True)
        cen = lin - mu
        var = jnp.mean(cen * cen, axis=0, keepdims=True)
        h2 = jnp.maximum(cen * lax.rsqrt(var + EPS) * g_ref[...] + be_ref[...],
                         0.0)
        ts_ref[...] = jnp.dot(h2, W3_ref[...],
                              preferred_element_type=jnp.float32)

    return pl.pallas_call(
        body,
        out_shape=jax.ShapeDtypeStruct((N, 2), jnp.float32),
    )(agg, deg, h, Wn, Ws, b, g, be, W3)


def _final_layer(agg3p, deg, s, b3):
    def body(agg3p_ref, deg_ref, s_ref, b3_ref, o_ref):
        deg = jnp.maximum(deg_ref[...], 1.0)
        agg3 = jnp.sum(agg3p_ref[...], axis=1, keepdims=True)
        lin = agg3 / deg + s_ref[...] + b3_ref[...]
        o_ref[...] = jax.nn.sigmoid(lin)

    return pl.pallas_call(
        body,
        out_shape=jax.ShapeDtypeStruct((N, 1), jnp.float32),
    )(agg3p, deg, s, b3)


def _agg_full(table):
    halves = table.reshape(NC, HROWS, D)[:, :HALF]
    return halves.reshape(NC * HALF, D)[:N]


def _deg_full(degh):
    halves = degh.reshape(NC, HROWS)[:, :HALF]
    return halves.reshape(NC * HALF)[:N].reshape(N, 1)


def kernel(x, edge_index, Wn1, Ws1, b1, g1, be1, Wn2, Ws2, b2, g2, be2,
           Wn3, Ws3, b3):
    src = edge_index[0].astype(jnp.int32)
    dst = edge_index[1].astype(jnp.int32)

    srcs_p, dsts_p, cnts_p = _sc_partition(src, dst)

    agg1_raw, deg_raw = _sc_agg_deg(x, srcs_p, dsts_p, cnts_p)
    agg1 = _agg_full(agg1_raw)
    deg = _deg_full(deg_raw)
    h1 = _dense_layer(agg1, deg, x, Wn1, Ws1, b1.reshape(1, D),
                      g1.reshape(1, D), be1.reshape(1, D))

    agg2_raw = _sc_agg_nodeg(h1, srcs_p, dsts_p, cnts_p)
    agg2 = _agg_full(agg2_raw)
    W3 = jnp.concatenate([Wn3, Ws3], axis=1)
    ts = _dense_layer2(agg2, deg, h1, Wn2, Ws2, b2.reshape(1, D),
                       g2.reshape(1, D), be2.reshape(1, D), W3)
    t = ts[:, 0:1].reshape(N)
    s = ts[:, 1:2]

    agg3p = _sc_agg_scalar(t, src, dst).reshape(NW, NPAD)[:, :N].T

    out = _final_layer(agg3p, deg, s, b3.reshape(1, 1))
    return out.reshape(N)
